# Initial kernel scaffold; baseline (speedup 1.0000x reference)
import jax
import jax.numpy as jnp
from jax.experimental import pallas as pl

N = 10000
F = 128
T = 20
K = 2


def _act_res_body(acc_ref, h_ref, b_ref, o_ref):
    a = acc_ref[...] + b_ref[...]
    o_ref[...] = jnp.where(a >= 0, a, 0.01 * a) + h_ref[...]


def kernel(x, edge_index, edge_weight, conv_w, conv_b, ln_gamma, ln_beta, tag_w, tag_b):
    xp = jnp.pad(x, ((0, 0), (0, 0), (2, 2)))
    y = jax.lax.conv_general_dilated(xp, conv_w, window_strides=(1,), padding='VALID',
                                     dimension_numbers=('NCH', 'OIH', 'NCH'))
    y = y + conv_b[None, :, None]
    y = y[:, :, :T]
    y = jax.nn.leaky_relu(y, 0.01)
    mu = y.mean(axis=(1, 2), keepdims=True)
    var = y.var(axis=(1, 2), keepdims=True)
    y = (y - mu) / jnp.sqrt(var + 1e-5) * ln_gamma[None] + ln_beta[None]

    row = edge_index[0]
    col = edge_index[1]
    deg = jax.ops.segment_sum(edge_weight, col, num_segments=N)
    dinv = jnp.where(deg > 0, jax.lax.rsqrt(deg), 0.0)
    norm = dinv[row] * edge_weight * dinv[col]

    h = y.reshape(N, F * T)
    p1 = jax.ops.segment_sum(h[row] * norm[:, None], col, num_segments=N)
    p2 = jax.ops.segment_sum(p1[row] * norm[:, None], col, num_segments=N)

    y3 = y
    p1 = p1.reshape(N, F, T)
    p2 = p2.reshape(N, F, T)
    acc = (jnp.einsum('nft,fg->ngt', y3, tag_w[0])
           + jnp.einsum('nft,fg->ngt', p1, tag_w[1])
           + jnp.einsum('nft,fg->ngt', p2, tag_w[2]))

    out = pl.pallas_call(
        _act_res_body,
        out_shape=jax.ShapeDtypeStruct((N, F, T), jnp.float32),
    )(acc, y3, jnp.broadcast_to(tag_b[None, :, None], (N, F, T)))
    return out


# speedup vs baseline: 1.5632x; 1.5632x over previous
import jax
import jax.numpy as jnp
from jax.experimental import pallas as pl

N = 10000
F = 128
T = 20
K = 2


def _act_res_body(acc_ref, h_ref, b_ref, o_ref):
    a = acc_ref[...] + b_ref[...]
    o_ref[...] = jnp.where(a >= 0, a, 0.01 * a) + h_ref[...]


def kernel(x, edge_index, edge_weight, conv_w, conv_b, ln_gamma, ln_beta, tag_w, tag_b):
    xp = jnp.pad(x, ((0, 0), (0, 0), (2, 2)))
    y = jax.lax.conv_general_dilated(xp, conv_w, window_strides=(1,), padding='VALID',
                                     dimension_numbers=('NCH', 'OIH', 'NCH'))
    y = y + conv_b[None, :, None]
    y = y[:, :, :T]
    y = jax.nn.leaky_relu(y, 0.01)
    mu = y.mean(axis=(1, 2), keepdims=True)
    var = y.var(axis=(1, 2), keepdims=True)
    y = (y - mu) / jnp.sqrt(var + 1e-5) * ln_gamma[None] + ln_beta[None]

    row = edge_index[0]
    col = edge_index[1]
    deg = jax.ops.segment_sum(edge_weight, col, num_segments=N)
    dinv = jnp.where(deg > 0, jax.lax.rsqrt(deg), 0.0)
    norm = dinv[row] * edge_weight * dinv[col]

    h = y.reshape(N, F * T)
    p1 = jax.ops.segment_sum(h[row] * norm[:, None], col, num_segments=N)
    p2 = jax.ops.segment_sum(p1[row] * norm[:, None], col, num_segments=N)

    y3 = y
    p1 = p1.reshape(N, F, T)
    p2 = p2.reshape(N, F, T)
    acc = (jnp.einsum('nft,fg->ngt', y3, tag_w[0])
           + jnp.einsum('nft,fg->ngt', p1, tag_w[1])
           + jnp.einsum('nft,fg->ngt', p2, tag_w[2]))

    blk = 200
    spec = pl.BlockSpec((blk, F * T), lambda i: (i, 0))
    out = pl.pallas_call(
        _act_res_body,
        grid=(N // blk,),
        in_specs=[spec, spec, spec],
        out_specs=spec,
        out_shape=jax.ShapeDtypeStruct((N, F * T), jnp.float32),
    )(acc.reshape(N, F * T), y3.reshape(N, F * T),
      jnp.broadcast_to(tag_b[None, :, None], (N, F, T)).reshape(N, F * T))
    return out.reshape(N, F, T)


# speedup vs baseline: 1.9332x; 1.2367x over previous
import functools

import jax
import jax.numpy as jnp
from jax import lax
from jax.experimental import pallas as pl
from jax.experimental.pallas import tpu as pltpu
from jax.experimental.pallas import tpu_sc as plsc

N = 10000
E = 160000
F = 128
T = 20
NTILES = 32
EPT = 5120
EP = NTILES * EPT
NBATCH = EPT // 128
NP = 10240
NSLICE = NP // 16

_f32 = jnp.float32
_mesh = plsc.VectorSubcoreMesh(core_axis_name="c", subcore_axis_name="s")
_sc_params = pltpu.CompilerParams(needs_layout_passes=False)


def _conv_ln_body(x_ref, w_ref, b_ref, g_ref, be_ref, o_ref):
    w0, w1, w2 = w_ref[0], w_ref[1], w_ref[2]
    b = b_ref[0]
    s = jnp.zeros((x_ref.shape[1], 1), _f32)
    s2 = jnp.zeros((x_ref.shape[1], 1), _f32)
    for t in range(T):
        z = (jnp.dot(x_ref[t], w0, preferred_element_type=_f32)
             + jnp.dot(x_ref[t + 1], w1, preferred_element_type=_f32)
             + jnp.dot(x_ref[t + 2], w2, preferred_element_type=_f32)
             + b[None, :])
        z = jnp.where(z >= 0, z, 0.01 * z)
        s = s + jnp.sum(z, axis=1, keepdims=True)
        s2 = s2 + jnp.sum(z * z, axis=1, keepdims=True)
        o_ref[t] = z
    mu = s * (1.0 / (F * T))
    var = s2 * (1.0 / (F * T)) - mu * mu
    r = lax.rsqrt(var + 1e-5)
    for t in range(T):
        o_ref[t] = (o_ref[t] - mu) * r * g_ref[t][None, :] + be_ref[t][None, :]


def _deg_body(col_hbm, w_hbm, out_hbm, acc, col_l, w_l):
    c = lax.axis_index("c")
    s = lax.axis_index("s")
    wid = c * 16 + s
    pltpu.sync_copy(col_hbm.at[pl.ds(wid * EPT, EPT)], col_l)
    pltpu.sync_copy(w_hbm.at[pl.ds(wid * EPT, EPT)], w_l)

    def zbody(i, _):
        acc[pl.ds(i * 16, 16)] = jnp.zeros((16,), _f32)
        return 0
    lax.fori_loop(0, NP // 16, zbody, 0)

    def ebody(i, _):
        plsc.addupdate_scatter(acc, [col_l[pl.ds(i * 16, 16)]],
                               w_l[pl.ds(i * 16, 16)])
        return 0
    lax.fori_loop(0, EPT // 16, ebody, 0)
    pltpu.sync_copy(acc, out_hbm.at[pl.ds(wid * NP, NP)])


def _dinv_body(d_ref, o_ref):
    deg = jnp.sum(d_ref[...], axis=0, keepdims=True)
    o_ref[...] = jnp.where(deg > 0, lax.rsqrt(deg), 0.0)


def _norm_body(dinv_hbm, row_hbm, col_hbm, w_hbm, out_hbm,
               dinv_l, row_l, col_l, w_l, norm_l):
    c = lax.axis_index("c")
    s = lax.axis_index("s")
    wid = c * 16 + s
    pltpu.sync_copy(dinv_hbm, dinv_l)
    pltpu.sync_copy(row_hbm.at[pl.ds(wid * EPT, EPT)], row_l)
    pltpu.sync_copy(col_hbm.at[pl.ds(wid * EPT, EPT)], col_l)
    pltpu.sync_copy(w_hbm.at[pl.ds(wid * EPT, EPT)], w_l)

    def ebody(i, _):
        rv = plsc.load_gather(dinv_l, [row_l[pl.ds(i * 16, 16)]])
        cv = plsc.load_gather(dinv_l, [col_l[pl.ds(i * 16, 16)]])
        norm_l[pl.ds(i * 16, 16)] = rv * w_l[pl.ds(i * 16, 16)] * cv
        return 0
    lax.fori_loop(0, EPT // 16, ebody, 0)
    pltpu.sync_copy(norm_l, out_hbm.at[pl.ds(wid * EPT, EPT)])


def _hop_body(ytab, row2, col2, norm2, parts, acc,
              row_l, col_l, norm_l, gidx, rbuf, zbuf, sem):
    c = lax.axis_index("c")
    s = lax.axis_index("s")
    wid = c * 16 + s
    pltpu.sync_copy(row2.at[pl.ds(wid * NBATCH, NBATCH)], row_l)
    pltpu.sync_copy(col2.at[pl.ds(wid * NBATCH, NBATCH)], col_l)
    pltpu.sync_copy(norm2.at[pl.ds(wid * NBATCH, NBATCH)], norm_l)

    def zb(i, _):
        for k in range(8):
            zbuf[i, pl.ds(k * 16, 16)] = jnp.zeros((16,), _f32)
        return 0
    lax.fori_loop(0, NSLICE // 5, zb, 0)

    def tbody(t, _):
        def zc(q, _):
            pltpu.sync_copy(zbuf, acc.at[pl.ds(s * NSLICE + q * (NSLICE // 5),
                                               NSLICE // 5)])
            return 0
        lax.fori_loop(0, 5, zc, 0)
        plsc.subcore_barrier()
        base = t * NP

        def jbody(j, _):
            for k in range(8):
                gidx[pl.ds(k * 16, 16)] = row_l[j, pl.ds(k * 16, 16)] + base
            pltpu.async_copy(ytab.at[gidx], rbuf, sem).wait()

            def ebody(g, _):
                nv = norm_l[j, pl.ds(g * 16, 16)]
                for k in range(16):
                    sc = nv[k]
                    r = g * 16 + k
                    for q in range(8):
                        rbuf[r, pl.ds(q * 16, 16)] = (
                            rbuf[r, pl.ds(q * 16, 16)] * sc)
                return 0
            lax.fori_loop(0, 8, ebody, 0)
            pltpu.sync_copy(rbuf, acc.at[col_l.at[j]], add=True)
            return 0
        lax.fori_loop(0, NBATCH, jbody, 0)
        plsc.subcore_barrier()
        off = (c * T + t) * NP + s * NSLICE
        pltpu.sync_copy(acc.at[pl.ds(s * NSLICE, NSLICE)],
                        parts.at[pl.ds(off, NSLICE)])
        plsc.subcore_barrier()
        return 0
    lax.fori_loop(0, T, tbody, 0)


_hop = functools.partial(
    pl.kernel,
    out_type=jax.ShapeDtypeStruct((2 * T * NP, F), _f32),
    mesh=_mesh,
    compiler_params=_sc_params,
    scratch_types=[
        pltpu.VMEM_SHARED((NP, F), _f32),
        pltpu.VMEM((NBATCH, 128), jnp.int32),
        pltpu.VMEM((NBATCH, 128), jnp.int32),
        pltpu.VMEM((NBATCH, 128), _f32),
        pltpu.VMEM((128,), jnp.int32),
        pltpu.VMEM((128, F), _f32),
        pltpu.VMEM((NSLICE // 5, F), _f32),
        pltpu.SemaphoreType.DMA,
    ],
)(_hop_body)


def _merge_body(a_ref, b_ref, o_ref):
    o_ref[...] = a_ref[...] + b_ref[...]


def _final_body(y_ref, p1_ref, p2a_ref, p2b_ref, w_ref, b_ref, o_ref):
    h = y_ref[0]
    a = (jnp.dot(h, w_ref[0], preferred_element_type=_f32)
         + jnp.dot(p1_ref[0], w_ref[1], preferred_element_type=_f32)
         + jnp.dot(p2a_ref[0] + p2b_ref[0], w_ref[2],
                   preferred_element_type=_f32)
         + b_ref[0][None, :])
    o_ref[0] = jnp.where(a >= 0, a, 0.01 * a) + h


def kernel(x, edge_index, edge_weight, conv_w, conv_b, ln_gamma, ln_beta,
           tag_w, tag_b):
    xT = jnp.pad(x.transpose(2, 0, 1), ((2, 0), (0, 0), (0, 0)))
    wT = conv_w.transpose(2, 1, 0)
    gT = ln_gamma.T
    beT = ln_beta.T
    pad = EP - E
    row = jnp.pad(edge_index[0], (0, pad))
    col = jnp.pad(edge_index[1], (0, pad))
    wgt = jnp.pad(edge_weight, (0, pad))
    row2 = row.reshape(EP // 128, 128)
    col2 = col.reshape(EP // 128, 128)

    Bn = 400
    y = pl.pallas_call(
        _conv_ln_body,
        grid=(N // Bn,),
        in_specs=[
            pl.BlockSpec((T + 2, Bn, F), lambda n: (0, n, 0)),
            pl.BlockSpec((3, F, F), lambda n: (0, 0, 0)),
            pl.BlockSpec((1, F), lambda n: (0, 0)),
            pl.BlockSpec((T, F), lambda n: (0, 0)),
            pl.BlockSpec((T, F), lambda n: (0, 0)),
        ],
        out_specs=pl.BlockSpec((T, Bn, F), lambda n: (0, n, 0)),
        out_shape=jax.ShapeDtypeStruct((T, N, F), _f32),
    )(xT, wT, conv_b.reshape(1, F), gT, beT)

    degparts = pl.kernel(
        _deg_body,
        out_type=jax.ShapeDtypeStruct((NTILES * NP,), _f32),
        mesh=_mesh,
        compiler_params=_sc_params,
        scratch_types=[
            pltpu.VMEM((NP,), _f32),
            pltpu.VMEM((EPT,), jnp.int32),
            pltpu.VMEM((EPT,), _f32),
        ],
    )(col, wgt)

    dinv = pl.pallas_call(
        _dinv_body,
        grid=(1,),
        in_specs=[pl.BlockSpec((NTILES, NP), lambda i: (0, 0))],
        out_specs=pl.BlockSpec((1, NP), lambda i: (0, 0)),
        out_shape=jax.ShapeDtypeStruct((1, NP), _f32),
    )(degparts.reshape(NTILES, NP)).reshape(NP)

    norm = pl.kernel(
        _norm_body,
        out_type=jax.ShapeDtypeStruct((EP,), _f32),
        mesh=_mesh,
        compiler_params=_sc_params,
        scratch_types=[
            pltpu.VMEM((NP,), _f32),
            pltpu.VMEM((EPT,), jnp.int32),
            pltpu.VMEM((EPT,), jnp.int32),
            pltpu.VMEM((EPT,), _f32),
            pltpu.VMEM((EPT,), _f32),
        ],
    )(dinv, row, col, wgt)
    norm2 = norm.reshape(EP // 128, 128)

    yp = jnp.pad(y, ((0, 0), (0, NP - N), (0, 0))).reshape(T * NP, F)
    parts1 = _hop(yp, row2, col2, norm2)
    p1 = pl.pallas_call(
        _merge_body,
        grid=(T * NP // 1024,),
        in_specs=[pl.BlockSpec((1024, F), lambda i: (i, 0)),
                  pl.BlockSpec((1024, F), lambda i: (i, 0))],
        out_specs=pl.BlockSpec((1024, F), lambda i: (i, 0)),
        out_shape=jax.ShapeDtypeStruct((T * NP, F), _f32),
    )(parts1[:T * NP], parts1[T * NP:])
    parts2 = _hop(p1, row2, col2, norm2)

    Bf = 1000
    out = pl.pallas_call(
        _final_body,
        grid=(T, N // Bf),
        in_specs=[
            pl.BlockSpec((1, Bf, F), lambda t, n: (t, n, 0)),
            pl.BlockSpec((1, Bf, F), lambda t, n: (t, n, 0)),
            pl.BlockSpec((1, Bf, F), lambda t, n: (t, n, 0)),
            pl.BlockSpec((1, Bf, F), lambda t, n: (t, n, 0)),
            pl.BlockSpec((3, F, F), lambda t, n: (0, 0, 0)),
            pl.BlockSpec((1, F), lambda t, n: (0, 0)),
        ],
        out_specs=pl.BlockSpec((1, Bf, F), lambda t, n: (t, n, 0)),
        out_shape=jax.ShapeDtypeStruct((T, N, F), _f32),
    )(y, p1.reshape(T, NP, F)[:, :N], parts2[:T * NP].reshape(T, NP, F)[:, :N],
      parts2[T * NP:].reshape(T, NP, F)[:, :N], tag_w, tag_b.reshape(1, F))
    return out.transpose(1, 2, 0)


# speedup vs baseline: 2.3407x; 1.2108x over previous
import functools

import jax
import jax.numpy as jnp
from jax import lax
from jax.experimental import pallas as pl
from jax.experimental.pallas import tpu as pltpu
from jax.experimental.pallas import tpu_sc as plsc

N = 10000
E = 160000
F = 128
T = 20
NTILES = 32
EPT = 5120
EP = NTILES * EPT
NBATCH = EPT // 128
NP = 10240
NSLICE = NP // 16

_f32 = jnp.float32
_mesh = plsc.VectorSubcoreMesh(core_axis_name="c", subcore_axis_name="s")
_sc_params = pltpu.CompilerParams(needs_layout_passes=False)


def _conv_ln_body(x_ref, w_ref, b_ref, g_ref, be_ref, o_ref):
    w0, w1, w2 = w_ref[0], w_ref[1], w_ref[2]
    b = b_ref[0]
    s = jnp.zeros((x_ref.shape[1], 1), _f32)
    s2 = jnp.zeros((x_ref.shape[1], 1), _f32)
    for t in range(T):
        z = (jnp.dot(x_ref[t], w0, preferred_element_type=_f32)
             + jnp.dot(x_ref[t + 1], w1, preferred_element_type=_f32)
             + jnp.dot(x_ref[t + 2], w2, preferred_element_type=_f32)
             + b[None, :])
        z = jnp.where(z >= 0, z, 0.01 * z)
        s = s + jnp.sum(z, axis=1, keepdims=True)
        s2 = s2 + jnp.sum(z * z, axis=1, keepdims=True)
        o_ref[t] = z
    mu = s * (1.0 / (F * T))
    var = s2 * (1.0 / (F * T)) - mu * mu
    r = lax.rsqrt(var + 1e-5)
    for t in range(T):
        o_ref[t] = (o_ref[t] - mu) * r * g_ref[t][None, :] + be_ref[t][None, :]


def _deg_body(col_hbm, w_hbm, out_hbm, acc, col_l, w_l):
    c = lax.axis_index("c")
    s = lax.axis_index("s")
    wid = c * 16 + s
    pltpu.sync_copy(col_hbm.at[pl.ds(wid * EPT, EPT)], col_l)
    pltpu.sync_copy(w_hbm.at[pl.ds(wid * EPT, EPT)], w_l)

    def zbody(i, _):
        acc[pl.ds(i * 16, 16)] = jnp.zeros((16,), _f32)
        return 0
    lax.fori_loop(0, NP // 16, zbody, 0)

    def ebody(i, _):
        plsc.addupdate_scatter(acc, [col_l[pl.ds(i * 16, 16)]],
                               w_l[pl.ds(i * 16, 16)])
        return 0
    lax.fori_loop(0, EPT // 16, ebody, 0)
    pltpu.sync_copy(acc, out_hbm.at[pl.ds(wid * NP, NP)])


def _dinv_body(d_ref, o_ref):
    deg = jnp.sum(d_ref[...], axis=0, keepdims=True)
    o_ref[...] = jnp.where(deg > 0, lax.rsqrt(deg), 0.0)


def _norm_body(dinv_hbm, row_hbm, col_hbm, w_hbm, out_hbm,
               dinv_l, row_l, col_l, w_l, norm_l):
    c = lax.axis_index("c")
    s = lax.axis_index("s")
    wid = c * 16 + s
    pltpu.sync_copy(dinv_hbm, dinv_l)
    pltpu.sync_copy(row_hbm.at[pl.ds(wid * EPT, EPT)], row_l)
    pltpu.sync_copy(col_hbm.at[pl.ds(wid * EPT, EPT)], col_l)
    pltpu.sync_copy(w_hbm.at[pl.ds(wid * EPT, EPT)], w_l)

    def ebody(i, _):
        rv = plsc.load_gather(dinv_l, [row_l[pl.ds(i * 16, 16)]])
        cv = plsc.load_gather(dinv_l, [col_l[pl.ds(i * 16, 16)]])
        norm_l[pl.ds(i * 16, 16)] = rv * w_l[pl.ds(i * 16, 16)] * cv
        return 0
    lax.fori_loop(0, EPT // 16, ebody, 0)
    pltpu.sync_copy(norm_l, out_hbm.at[pl.ds(wid * EPT, EPT)])


def _hop_body(ytab, row2, col2, norm2, parts, acc,
              row_l, col_l, norm_l, gidx_a, gidx_b, rbuf_a, rbuf_b,
              sem_a, sem_b):
    c = lax.axis_index("c")
    s = lax.axis_index("s")
    wid = c * 16 + s
    pltpu.sync_copy(row2.at[pl.ds(wid * NBATCH, NBATCH)], row_l)
    pltpu.sync_copy(col2.at[pl.ds(wid * NBATCH, NBATCH)], col_l)
    pltpu.sync_copy(norm2.at[pl.ds(wid * NBATCH, NBATCH)], norm_l)

    def tbody(t, _):
        def zb(i, _):
            for k in range(8):
                rbuf_a[i, pl.ds(k * 16, 16)] = jnp.zeros((16,), _f32)
            return 0
        lax.fori_loop(0, 128, zb, 0)

        def zc(q, _):
            pltpu.sync_copy(rbuf_a, acc.at[pl.ds(s * NSLICE + q * 128, 128)])
            return 0
        lax.fori_loop(0, 5, zc, 0)
        plsc.subcore_barrier()
        base = t * NP

        def fire(j, gidx, rbuf, sem):
            for k in range(8):
                gidx[pl.ds(k * 16, 16)] = row_l[j, pl.ds(k * 16, 16)] + base
            pltpu.async_copy(ytab.at[gidx], rbuf, sem)

        def scale_scatter(j, rbuf):
            def ebody(g, _):
                nv = norm_l[j, pl.ds(g * 16, 16)]
                for k in range(16):
                    sc = nv[k]
                    r = g * 16 + k
                    for q in range(8):
                        rbuf[r, pl.ds(q * 16, 16)] = (
                            rbuf[r, pl.ds(q * 16, 16)] * sc)
                return 0
            lax.fori_loop(0, 8, ebody, 0)
            pltpu.sync_copy(rbuf, acc.at[col_l.at[j]], add=True)

        fire(0, gidx_a, rbuf_a, sem_a)

        def pair(p, _):
            j0 = p * 2
            fire(j0 + 1, gidx_b, rbuf_b, sem_b)
            pltpu.make_async_copy(ytab.at[gidx_a], rbuf_a, sem_a).wait()
            scale_scatter(j0, rbuf_a)

            @pl.when(p < NBATCH // 2 - 1)
            def _():
                fire(j0 + 2, gidx_a, rbuf_a, sem_a)
            pltpu.make_async_copy(ytab.at[gidx_b], rbuf_b, sem_b).wait()
            scale_scatter(j0 + 1, rbuf_b)
            return 0
        lax.fori_loop(0, NBATCH // 2, pair, 0)
        plsc.subcore_barrier()
        off = (c * T + t) * NP + s * NSLICE
        pltpu.sync_copy(acc.at[pl.ds(s * NSLICE, NSLICE)],
                        parts.at[pl.ds(off, NSLICE)])
        plsc.subcore_barrier()
        return 0
    lax.fori_loop(0, T, tbody, 0)


_hop = functools.partial(
    pl.kernel,
    out_type=jax.ShapeDtypeStruct((2 * T * NP, F), _f32),
    mesh=_mesh,
    compiler_params=_sc_params,
    scratch_types=[
        pltpu.VMEM_SHARED((NP, F), _f32),
        pltpu.VMEM((NBATCH, 128), jnp.int32),
        pltpu.VMEM((NBATCH, 128), jnp.int32),
        pltpu.VMEM((NBATCH, 128), _f32),
        pltpu.VMEM((128,), jnp.int32),
        pltpu.VMEM((128,), jnp.int32),
        pltpu.VMEM((128, F), _f32),
        pltpu.VMEM((128, F), _f32),
        pltpu.SemaphoreType.DMA,
        pltpu.SemaphoreType.DMA,
    ],
)(_hop_body)


def _merge_body(a_ref, b_ref, o_ref):
    o_ref[...] = a_ref[...] + b_ref[...]


def _final_body(y_ref, p1_ref, p2a_ref, p2b_ref, w_ref, b_ref, o_ref):
    h = y_ref[0]
    a = (jnp.dot(h, w_ref[0], preferred_element_type=_f32)
         + jnp.dot(p1_ref[0], w_ref[1], preferred_element_type=_f32)
         + jnp.dot(p2a_ref[0] + p2b_ref[0], w_ref[2],
                   preferred_element_type=_f32)
         + b_ref[0][None, :])
    o_ref[0] = jnp.where(a >= 0, a, 0.01 * a) + h


def kernel(x, edge_index, edge_weight, conv_w, conv_b, ln_gamma, ln_beta,
           tag_w, tag_b):
    xT = jnp.pad(x.transpose(2, 0, 1), ((2, 0), (0, 0), (0, 0)))
    wT = conv_w.transpose(2, 1, 0)
    gT = ln_gamma.T
    beT = ln_beta.T
    pad = EP - E
    row = jnp.pad(edge_index[0], (0, pad))
    col = jnp.pad(edge_index[1], (0, pad))
    wgt = jnp.pad(edge_weight, (0, pad))
    row2 = row.reshape(EP // 128, 128)
    col2 = col.reshape(EP // 128, 128)

    Bn = 400
    y = pl.pallas_call(
        _conv_ln_body,
        grid=(N // Bn,),
        in_specs=[
            pl.BlockSpec((T + 2, Bn, F), lambda n: (0, n, 0)),
            pl.BlockSpec((3, F, F), lambda n: (0, 0, 0)),
            pl.BlockSpec((1, F), lambda n: (0, 0)),
            pl.BlockSpec((T, F), lambda n: (0, 0)),
            pl.BlockSpec((T, F), lambda n: (0, 0)),
        ],
        out_specs=pl.BlockSpec((T, Bn, F), lambda n: (0, n, 0)),
        out_shape=jax.ShapeDtypeStruct((T, N, F), _f32),
    )(xT, wT, conv_b.reshape(1, F), gT, beT)

    degparts = pl.kernel(
        _deg_body,
        out_type=jax.ShapeDtypeStruct((NTILES * NP,), _f32),
        mesh=_mesh,
        compiler_params=_sc_params,
        scratch_types=[
            pltpu.VMEM((NP,), _f32),
            pltpu.VMEM((EPT,), jnp.int32),
            pltpu.VMEM((EPT,), _f32),
        ],
    )(col, wgt)

    dinv = pl.pallas_call(
        _dinv_body,
        grid=(1,),
        in_specs=[pl.BlockSpec((NTILES, NP), lambda i: (0, 0))],
        out_specs=pl.BlockSpec((1, NP), lambda i: (0, 0)),
        out_shape=jax.ShapeDtypeStruct((1, NP), _f32),
    )(degparts.reshape(NTILES, NP)).reshape(NP)

    norm = pl.kernel(
        _norm_body,
        out_type=jax.ShapeDtypeStruct((EP,), _f32),
        mesh=_mesh,
        compiler_params=_sc_params,
        scratch_types=[
            pltpu.VMEM((NP,), _f32),
            pltpu.VMEM((EPT,), jnp.int32),
            pltpu.VMEM((EPT,), jnp.int32),
            pltpu.VMEM((EPT,), _f32),
            pltpu.VMEM((EPT,), _f32),
        ],
    )(dinv, row, col, wgt)
    norm2 = norm.reshape(EP // 128, 128)

    yp = jnp.pad(y, ((0, 0), (0, NP - N), (0, 0))).reshape(T * NP, F)
    parts1 = _hop(yp, row2, col2, norm2)
    p1 = pl.pallas_call(
        _merge_body,
        grid=(T * NP // 1024,),
        in_specs=[pl.BlockSpec((1024, F), lambda i: (i, 0)),
                  pl.BlockSpec((1024, F), lambda i: (i, 0))],
        out_specs=pl.BlockSpec((1024, F), lambda i: (i, 0)),
        out_shape=jax.ShapeDtypeStruct((T * NP, F), _f32),
    )(parts1[:T * NP], parts1[T * NP:])
    parts2 = _hop(p1, row2, col2, norm2)

    Bf = 1000
    out = pl.pallas_call(
        _final_body,
        grid=(T, N // Bf),
        in_specs=[
            pl.BlockSpec((1, Bf, F), lambda t, n: (t, n, 0)),
            pl.BlockSpec((1, Bf, F), lambda t, n: (t, n, 0)),
            pl.BlockSpec((1, Bf, F), lambda t, n: (t, n, 0)),
            pl.BlockSpec((1, Bf, F), lambda t, n: (t, n, 0)),
            pl.BlockSpec((3, F, F), lambda t, n: (0, 0, 0)),
            pl.BlockSpec((1, F), lambda t, n: (0, 0)),
        ],
        out_specs=pl.BlockSpec((1, Bf, F), lambda t, n: (t, n, 0)),
        out_shape=jax.ShapeDtypeStruct((T, N, F), _f32),
    )(y, p1.reshape(T, NP, F)[:, :N], parts2[:T * NP].reshape(T, NP, F)[:, :N],
      parts2[T * NP:].reshape(T, NP, F)[:, :N], tag_w, tag_b.reshape(1, F))
    return out.transpose(1, 2, 0)
